# Initial kernel scaffold; baseline (speedup 1.0000x reference)
import jax
import jax.numpy as jnp
from jax.experimental import pallas as pl


def kernel(inputs):
    raise NotImplementedError("write your pallas kernel here")


# speedup vs baseline: 11.8517x; 11.8517x over previous
import jax
import jax.numpy as jnp
from jax.experimental import pallas as pl
from jax.experimental.pallas import tpu as pltpu

_K = 32
_ITERS = 10
_N = 224 * 224


_ROWS = _N // 128


def _kmeans_body(x_ref, c_ref, o_ref):
    xp = x_ref[0]
    x0 = x_ref[0, 0:1, :].reshape(_ROWS, 128)
    x1 = x_ref[0, 1:2, :].reshape(_ROWS, 128)
    x2 = x_ref[0, 2:3, :].reshape(_ROWS, 128)
    xsq1 = x_ref[0, 0:1, :] * x_ref[0, 0:1, :]
    xsq2 = x_ref[0, 1:2, :] * x_ref[0, 1:2, :]
    xsq3 = x_ref[0, 2:3, :] * x_ref[0, 2:3, :]
    xsq = (xsq1 + xsq2) + xsq3
    xb = xp.astype(jnp.bfloat16)
    c_init = c_ref[0]

    def one_iter(_, c):
        csq = jnp.sum(c * c, axis=1)[:, None]
        cb = c.astype(jnp.bfloat16)
        t = jax.lax.dot_general(
            cb, xb, (((1,), (0,)), ((), ())),
            preferred_element_type=jnp.float32,
        )
        dists = (csq + xsq) - 2.0 * t
        arg = jnp.argmin(dists, axis=0).reshape(_ROWS, 128)
        rows = []
        for k in range(_K):
            m = arg == k
            cnt = jnp.sum(m.astype(jnp.float32))
            s0 = jnp.sum(jnp.where(m, x0, 0.0))
            s1 = jnp.sum(jnp.where(m, x1, 0.0))
            s2 = jnp.sum(jnp.where(m, x2, 0.0))
            z = jnp.float32(0.0)
            rows.append(jnp.stack([s0 / cnt, s1 / cnt, s2 / cnt, z, z, z, z, z]))
        return jnp.stack(rows)

    c = jax.lax.fori_loop(0, _ITERS, one_iter, c_init)
    o_ref[0] = c


def _run(xp, c_init):
    return pl.pallas_call(
        _kmeans_body,
        grid=(8,),
        in_specs=[
            pl.BlockSpec((1, 8, _N), lambda b: (b, 0, 0)),
            pl.BlockSpec((1, _K, 8), lambda b: (b, 0, 0)),
        ],
        out_specs=pl.BlockSpec((1, _K, 8), lambda b: (b, 0, 0)),
        out_shape=jax.ShapeDtypeStruct((8, _K, 8), jnp.float32),
        compiler_params=pltpu.CompilerParams(
            dimension_semantics=("parallel",),
        ),
    )(xp, c_init)


def kernel(inputs):
    inputs = inputs.astype(jnp.float32)
    B = inputs.shape[0]
    x = inputs.reshape(B, -1, 3)
    N = x.shape[1]
    keys = jax.random.split(jax.random.key(42), B)
    idx = jax.vmap(lambda k: jax.random.permutation(k, N)[:_K])(keys)
    c_init = jnp.take_along_axis(x, idx[:, :, None], axis=1)
    c_init = jnp.pad(c_init, ((0, 0), (0, 0), (0, 5)))
    xp = jnp.pad(x.transpose(0, 2, 1), ((0, 0), (0, 5), (0, 0)))
    c = _run(xp, c_init)
    return c[:, :, :3].reshape(B, _K * 3)


# speedup vs baseline: 27.5064x; 2.3209x over previous
import jax
import jax.numpy as jnp
from jax.experimental import pallas as pl
from jax.experimental.pallas import tpu as pltpu

_K = 32
_ITERS = 10
_N = 224 * 224


_ROWS = _N // 128


def _kmeans_body(x_ref, c_ref, o_ref):
    xp = x_ref[0]
    x0 = x_ref[0, 0:1, :].reshape(_ROWS, 128)
    x1 = x_ref[0, 1:2, :].reshape(_ROWS, 128)
    x2 = x_ref[0, 2:3, :].reshape(_ROWS, 128)
    xsq1 = x_ref[0, 0:1, :] * x_ref[0, 0:1, :]
    xsq2 = x_ref[0, 1:2, :] * x_ref[0, 1:2, :]
    xsq3 = x_ref[0, 2:3, :] * x_ref[0, 2:3, :]
    xsq = (xsq1 + xsq2) + xsq3
    xb = xp.astype(jnp.bfloat16)
    c_init = c_ref[0]

    def one_iter(_, c):
        csq = jnp.sum(c * c, axis=1)[:, None]
        cb = c.astype(jnp.bfloat16)
        t = jax.lax.dot_general(
            cb, xb, (((1,), (0,)), ((), ())),
            preferred_element_type=jnp.float32,
        )
        dists = (csq + xsq) - 2.0 * t
        arg = jnp.argmin(dists, axis=0).reshape(_ROWS, 128)
        rows = []
        for k in range(_K):
            m = arg == k
            cnt = jnp.sum(m.astype(jnp.float32))
            s0 = jnp.sum(jnp.where(m, x0, 0.0))
            s1 = jnp.sum(jnp.where(m, x1, 0.0))
            s2 = jnp.sum(jnp.where(m, x2, 0.0))
            z = jnp.float32(0.0)
            rows.append(jnp.stack([s0 / cnt, s1 / cnt, s2 / cnt, z, z, z, z, z]))
        return jnp.stack(rows)

    c = jax.lax.fori_loop(0, _ITERS, one_iter, c_init)
    o_ref[0] = c


def _run(xp, c_init):
    return pl.pallas_call(
        _kmeans_body,
        grid=(8,),
        in_specs=[
            pl.BlockSpec((1, 8, _N), lambda b: (b, 0, 0)),
            pl.BlockSpec((1, _K, 8), lambda b: (b, 0, 0)),
        ],
        out_specs=pl.BlockSpec((1, _K, 8), lambda b: (b, 0, 0)),
        out_shape=jax.ShapeDtypeStruct((8, _K, 8), jnp.float32),
        compiler_params=pltpu.CompilerParams(
            dimension_semantics=("parallel",),
        ),
    )(xp, c_init)


import functools
import numpy as np


@functools.lru_cache(maxsize=None)
def _init_indices(B, N):
    with jax.ensure_compile_time_eval():
        with jax.default_device(jax.local_devices(backend="cpu")[0]):
            keys = jax.random.split(jax.random.key(42), B)
            idx = jax.vmap(lambda k: jax.random.permutation(k, N)[:_K])(keys)
            return np.asarray(idx)


def kernel(inputs):
    inputs = inputs.astype(jnp.float32)
    B = inputs.shape[0]
    x = inputs.reshape(B, -1, 3)
    N = x.shape[1]
    idx = jnp.asarray(_init_indices(B, N))
    c_init = jnp.take_along_axis(x, idx[:, :, None], axis=1)
    c_init = jnp.pad(c_init, ((0, 0), (0, 0), (0, 5)))
    xp = jnp.pad(x.transpose(0, 2, 1), ((0, 0), (0, 5), (0, 0)))
    c = _run(xp, c_init)
    return c[:, :, :3].reshape(B, _K * 3)
